# baseline (speedup 1.0000x reference)
import functools

import jax
import jax.numpy as jnp
from jax import lax
from jax.experimental import pallas as pl
from jax.experimental.pallas import tpu as pltpu
from jax.experimental.pallas import tpu_sc as plsc


def kernel(x, token_table, pos_table):
    B, S = x.shape
    V, D = token_table.shape
    N = B * S
    L = 16

    info = plsc.get_sparse_core_info()
    NC, NS = info.num_cores, info.num_subcores
    NW = NC * NS
    b_per_w = N // NW
    H = b_per_w // 2
    BLK_PER_CORE = NS // B
    POS_PER_CORE = BLK_PER_CORE * b_per_w
    assert N % NW == 0 and D % L == 0 and H <= 128 and H % 8 == 0
    assert NS % B == 0 and S == NC * POS_PER_CORE

    mesh = plsc.VectorSubcoreMesh(core_axis_name="c", subcore_axis_name="s")

    @functools.partial(
        pl.kernel,
        mesh=mesh,
        out_type=jax.ShapeDtypeStruct((B, S, D), jnp.float32),
        scratch_types=[
            pltpu.VMEM((b_per_w,), jnp.int32),
            pltpu.VMEM((b_per_w, D), jnp.float32),
            pltpu.VMEM((b_per_w, D), jnp.float32),
            pltpu.VMEM_SHARED((POS_PER_CORE, D), jnp.float32),
            pltpu.SemaphoreType.DMA,
            pltpu.SemaphoreType.DMA,
            pltpu.SemaphoreType.DMA,
            pltpu.SemaphoreType.DMA,
            pltpu.SemaphoreType.DMA,
        ],
    )
    def sc_kernel(x_hbm, tok_hbm, pos_hbm, out_hbm, idx_v, pos_v, rows_v,
                  pos_sh, sem_p, sem_g0, sem_g1, sem_w0, sem_w1):
        c = lax.axis_index("c")
        s = lax.axis_index("s")
        b_idx = s // BLK_PER_CORE
        s_base = c * POS_PER_CORE + lax.rem(s, BLK_PER_CORE) * b_per_w

        pltpu.sync_copy(x_hbm.at[b_idx, pl.ds(s_base, b_per_w)], idx_v)
        g0 = pltpu.async_copy(
            tok_hbm.at[idx_v.at[pl.ds(0, H)]], rows_v.at[pl.ds(0, H)], sem_g0)
        g1 = pltpu.async_copy(
            tok_hbm.at[idx_v.at[pl.ds(H, H)]], rows_v.at[pl.ds(H, H)], sem_g1)

        half = POS_PER_CORE // 2

        @pl.when(s < 2)
        def _load_pos():
            pltpu.sync_copy(
                pos_hbm.at[pl.ds(c * POS_PER_CORE + s * half, half)],
                pos_sh.at[pl.ds(s * half, half)])

        plsc.subcore_barrier()
        p_cp = pltpu.async_copy(
            pos_sh.at[pl.ds(lax.rem(s, BLK_PER_CORE) * b_per_w, b_per_w)],
            pos_v, sem_p)

        p_cp.wait()
        g0.wait()

        @plsc.parallel_loop(0, H)
        def add0(i):
            for j in range(D // L):
                sl = pl.ds(j * L, L)
                rows_v[i, sl] = rows_v[i, sl] + pos_v[i, sl]

        w0 = pltpu.async_copy(
            rows_v.at[pl.ds(0, H)],
            out_hbm.at[b_idx, pl.ds(s_base, H)], sem_w0)
        g1.wait()

        @plsc.parallel_loop(H, b_per_w)
        def add1(i):
            for j in range(D // L):
                sl = pl.ds(j * L, L)
                rows_v[i, sl] = rows_v[i, sl] + pos_v[i, sl]

        w1 = pltpu.async_copy(
            rows_v.at[pl.ds(H, H)],
            out_hbm.at[b_idx, pl.ds(s_base + H, H)], sem_w1)
        w0.wait()
        w1.wait()

    return sc_kernel(x, token_table, pos_table)


# speedup vs baseline: 1.0098x; 1.0098x over previous
import functools

import jax
import jax.numpy as jnp
from jax import lax
from jax.experimental import pallas as pl
from jax.experimental.pallas import tpu as pltpu
from jax.experimental.pallas import tpu_sc as plsc


def kernel(x, token_table, pos_table):
    B, S = x.shape
    V, D = token_table.shape
    N = B * S
    L = 16

    info = plsc.get_sparse_core_info()
    NC, NS = info.num_cores, info.num_subcores
    NW = NC * NS
    b_per_w = N // NW
    H = b_per_w // 2
    BLK_PER_CORE = NS // B
    POS_PER_CORE = BLK_PER_CORE * b_per_w
    assert N % NW == 0 and D % L == 0 and H <= 128 and H % 8 == 0
    assert NS % B == 0 and S == NC * POS_PER_CORE

    mesh = plsc.VectorSubcoreMesh(core_axis_name="c", subcore_axis_name="s")

    @functools.partial(
        pl.kernel,
        mesh=mesh,
        out_type=jax.ShapeDtypeStruct((B, S, D), jnp.float32),
        scratch_types=[
            pltpu.VMEM((b_per_w,), jnp.int32),
            pltpu.VMEM((b_per_w, D), jnp.float32),
            pltpu.VMEM((b_per_w, D), jnp.float32),
            pltpu.VMEM_SHARED((POS_PER_CORE, D), jnp.float32),
            pltpu.SemaphoreType.DMA,
            pltpu.SemaphoreType.DMA,
            pltpu.SemaphoreType.DMA,
            pltpu.SemaphoreType.DMA,
            pltpu.SemaphoreType.DMA,
        ],
    )
    def sc_kernel(x_hbm, tok_hbm, pos_hbm, out_hbm, idx_v, pos_v, rows_v,
                  pos_sh, sem_p, sem_g0, sem_g1, sem_w0, sem_w1):
        c = lax.axis_index("c")
        s = lax.axis_index("s")
        b_idx = s // BLK_PER_CORE
        s_base = c * POS_PER_CORE + lax.rem(s, BLK_PER_CORE) * b_per_w

        pltpu.sync_copy(x_hbm.at[b_idx, pl.ds(s_base, b_per_w)], idx_v)
        g0 = pltpu.async_copy(
            tok_hbm.at[idx_v.at[pl.ds(0, H)]], rows_v.at[pl.ds(0, H)], sem_g0)
        g1 = pltpu.async_copy(
            tok_hbm.at[idx_v.at[pl.ds(H, H)]], rows_v.at[pl.ds(H, H)], sem_g1)

        half = POS_PER_CORE // 2

        @pl.when(s < 2)
        def _load_pos():
            pltpu.sync_copy(
                pos_hbm.at[pl.ds(c * POS_PER_CORE + s * half, half)],
                pos_sh.at[pl.ds(s * half, half)])

        plsc.subcore_barrier()
        p_cp = pltpu.async_copy(
            pos_sh.at[pl.ds(lax.rem(s, BLK_PER_CORE) * b_per_w, b_per_w)],
            pos_v, sem_p)

        p_cp.wait()
        g0.wait()

        @plsc.parallel_loop(0, H)
        def add0(i):
            for j in range(D // L):
                sl = pl.ds(j * L, L)
                plsc.addupdate(rows_v.at[i, sl], pos_v[i, sl])

        w0 = pltpu.async_copy(
            rows_v.at[pl.ds(0, H)],
            out_hbm.at[b_idx, pl.ds(s_base, H)], sem_w0)
        g1.wait()

        @plsc.parallel_loop(H, b_per_w)
        def add1(i):
            for j in range(D // L):
                sl = pl.ds(j * L, L)
                plsc.addupdate(rows_v.at[i, sl], pos_v[i, sl])

        w1 = pltpu.async_copy(
            rows_v.at[pl.ds(H, H)],
            out_hbm.at[b_idx, pl.ds(s_base + H, H)], sem_w1)
        w0.wait()
        w1.wait()

    return sc_kernel(x, token_table, pos_table)
